# Initial kernel scaffold; baseline (speedup 1.0000x reference)
import jax
import jax.numpy as jnp
from jax.experimental import pallas as pl


def kernel(x, top_k, Wqkv, bqkv):
    raise NotImplementedError("write your pallas kernel here")


# speedup vs baseline: 4.4127x; 4.4127x over previous
import functools

import jax
import jax.numpy as jnp
from jax import lax
from jax.experimental import pallas as pl
from jax.experimental.pallas import tpu as pltpu
from jax.experimental.pallas import tpu_sc as plsc

HD = 64
PS = 2
B = 2
C = 768
H = 224
W = 224
NH = C // HD
PH = H // PS
PW = W // PS
KSEL = 512
NTOK = KSEL * PS * PS
NBN = B * NH
NPLANES = B * C
PLANE = H * W
SCALE = HD ** -0.5

NC = 2
NS = 16
NW = NC * NS
DPW = HD // NW

_MESH = plsc.VectorSubcoreMesh(core_axis_name="c", subcore_axis_name="s")


def _compute_positions(topk_v, pos_v):
    @pl.loop(0, KSEL // 16)
    def _pos_loop(ci):
        kv = topk_v[pl.ds(ci * 16, 16)]
        i = lax.div(kv, jnp.full((16,), PW, jnp.int32))
        j = kv - i * PW
        base = i * (PS * W) + j * PS
        pos_v[pl.ds(ci * 16, 16)] = base
        pos_v[pl.ds(KSEL + ci * 16, 16)] = base + 1
        pos_v[pl.ds(2 * KSEL + ci * 16, 16)] = base + W
        pos_v[pl.ds(3 * KSEL + ci * 16, 16)] = base + W + 1


@functools.partial(
    pl.kernel,
    out_type=jax.ShapeDtypeStruct((NPLANES, NTOK), jnp.float32),
    mesh=_MESH,
    compiler_params=pltpu.CompilerParams(needs_layout_passes=False),
    scratch_types=[
        pltpu.VMEM((KSEL,), jnp.int32),
        pltpu.VMEM((NTOK,), jnp.int32),
        pltpu.VMEM((PLANE,), jnp.float32),
        pltpu.VMEM((NTOK,), jnp.float32),
    ],
)
def _sc_gather(x_hbm, topk_hbm, tok_hbm, topk_v, pos_v, plane_v, tok_v):
    wid = lax.axis_index("s") * NC + lax.axis_index("c")

    @pl.loop(0, NBN)
    def _bn_loop(bn):
        pltpu.sync_copy(topk_hbm.at[bn], topk_v)
        _compute_positions(topk_v, pos_v)

        @pl.loop(0, DPW)
        def _plane_loop(local):
            g = bn * HD + wid * DPW + local
            pltpu.sync_copy(x_hbm.at[g], plane_v)

            @pl.loop(0, NTOK // 16)
            def _tok_loop(c2):
                idxv = pos_v[pl.ds(c2 * 16, 16)]
                tok_v[pl.ds(c2 * 16, 16)] = plsc.load_gather(plane_v, [idxv])

            pltpu.sync_copy(tok_v, tok_hbm.at[g])


@functools.partial(
    pl.kernel,
    out_type=jax.ShapeDtypeStruct((NPLANES, PLANE), jnp.float32),
    mesh=_MESH,
    compiler_params=pltpu.CompilerParams(needs_layout_passes=False),
    scratch_types=[
        pltpu.VMEM((KSEL,), jnp.int32),
        pltpu.VMEM((NTOK,), jnp.int32),
        pltpu.VMEM((PLANE,), jnp.float32),
        pltpu.VMEM((NTOK,), jnp.float32),
    ],
)
def _sc_scatter(outtok_hbm, topk_hbm, out_hbm, topk_v, pos_v, plane_v, tok_v):
    wid = lax.axis_index("s") * NC + lax.axis_index("c")

    @pl.loop(0, PLANE // 16)
    def _zero_loop(ci):
        plane_v[pl.ds(ci * 16, 16)] = jnp.zeros((16,), jnp.float32)

    @pl.loop(0, NBN)
    def _bn_loop(bn):
        pltpu.sync_copy(topk_hbm.at[bn], topk_v)
        _compute_positions(topk_v, pos_v)

        @pl.loop(0, DPW)
        def _plane_loop(local):
            g = bn * HD + wid * DPW + local
            pltpu.sync_copy(outtok_hbm.at[g], tok_v)

            @pl.loop(0, NTOK // 16)
            def _scat_loop(c2):
                idxv = pos_v[pl.ds(c2 * 16, 16)]
                plsc.store_scatter(plane_v, [idxv], tok_v[pl.ds(c2 * 16, 16)])

            pltpu.sync_copy(plane_v, out_hbm.at[g])

            @pl.loop(0, NTOK // 16)
            def _restore_loop(c2):
                idxv = pos_v[pl.ds(c2 * 16, 16)]
                plsc.store_scatter(
                    plane_v, [idxv], jnp.zeros((16,), jnp.float32)
                )


def _attn_body(tok_ref, w_ref, b_ref, out_ref):
    x = tok_ref[0]
    wq = w_ref[...]
    bias = b_ref[...]
    qkv = jnp.dot(wq, x, preferred_element_type=jnp.float32) + bias
    q = qkv[0:HD]
    k = qkv[HD:2 * HD]
    v = qkv[2 * HD:3 * HD]
    logits = lax.dot_general(
        q, k, (((0,), (0,)), ((), ())), preferred_element_type=jnp.float32
    ) * SCALE
    m = jnp.max(logits, axis=1, keepdims=True)
    p = jnp.exp(logits - m)
    s = jnp.sum(p, axis=1, keepdims=True)
    attn = p / s
    out_ref[0] = lax.dot_general(
        v, attn, (((1,), (1,)), ((), ())), preferred_element_type=jnp.float32
    )


_attn = pl.pallas_call(
    _attn_body,
    grid=(NBN,),
    in_specs=[
        pl.BlockSpec((1, HD, NTOK), lambda i: (i, 0, 0)),
        pl.BlockSpec((3 * HD, HD), lambda i: (0, 0)),
        pl.BlockSpec((3 * HD, 1), lambda i: (0, 0)),
    ],
    out_specs=pl.BlockSpec((1, HD, NTOK), lambda i: (i, 0, 0)),
    out_shape=jax.ShapeDtypeStruct((NBN, HD, NTOK), jnp.float32),
)


def kernel(x, top_k, Wqkv, bqkv):
    xf = x.reshape(NPLANES, PLANE)
    tk = top_k.reshape(NBN, KSEL)
    MODE = 3
    if MODE == 1:
        toks = _sc_gather(xf, tk)
        s = jnp.sum(toks) + jnp.sum(Wqkv) + jnp.sum(bqkv)
        return jnp.broadcast_to(s, (B, C, H, W))
    if MODE == 2:
        out = _sc_scatter(xf[:, :NTOK] * 1.0, tk)
        s = jnp.sum(out) + jnp.sum(Wqkv) + jnp.sum(bqkv)
        return jnp.broadcast_to(s, (B, C, H, W))
    toks = _sc_gather(xf, tk)
    out_t = _attn(toks.reshape(NBN, HD, NTOK), Wqkv, bqkv.reshape(3 * HD, 1))
    out = _sc_scatter(out_t.reshape(NPLANES, NTOK), tk)
    return out.reshape(B, C, H, W)


# speedup vs baseline: 6.1448x; 1.3925x over previous
import functools

import jax
import jax.numpy as jnp
from jax import lax
from jax.experimental import pallas as pl
from jax.experimental.pallas import tpu as pltpu
from jax.experimental.pallas import tpu_sc as plsc

HD = 64
PS = 2
B = 2
C = 768
H = 224
W = 224
NH = C // HD
PH = H // PS
PW = W // PS
KSEL = 512
NTOK = KSEL * PS * PS
NBN = B * NH
SCALE = HD ** -0.5

NC = 2
NS = 16
NW = NC * NS
DPW = HD // NW

_MESH = plsc.VectorSubcoreMesh(core_axis_name="c", subcore_axis_name="s")
_SC_PARAMS = pltpu.CompilerParams(needs_layout_passes=False)


def _compute_positions(topk_v, posi_v, posj_v):
    @pl.loop(0, KSEL // 16)
    def _pos_loop(ci):
        kv = topk_v[pl.ds(ci * 16, 16)]
        i = lax.div(kv, jnp.full((16,), PW, jnp.int32))
        j = kv - i * PW
        i2 = i * PS
        j2 = j * PS
        posi_v[pl.ds(ci * 16, 16)] = i2
        posj_v[pl.ds(ci * 16, 16)] = j2
        posi_v[pl.ds(KSEL + ci * 16, 16)] = i2
        posj_v[pl.ds(KSEL + ci * 16, 16)] = j2 + 1
        posi_v[pl.ds(2 * KSEL + ci * 16, 16)] = i2 + 1
        posj_v[pl.ds(2 * KSEL + ci * 16, 16)] = j2
        posi_v[pl.ds(3 * KSEL + ci * 16, 16)] = i2 + 1
        posj_v[pl.ds(3 * KSEL + ci * 16, 16)] = j2 + 1


def _plane_coords(bn, wid, local):
    b = lax.div(bn, NH)
    n = bn - b * NH
    ch = n * HD + wid * DPW + local
    return b, ch


@functools.partial(
    pl.kernel,
    out_type=jax.ShapeDtypeStruct((NBN, HD, NTOK), jnp.float32),
    mesh=_MESH,
    compiler_params=_SC_PARAMS,
    scratch_types=[
        pltpu.VMEM((KSEL,), jnp.int32),
        pltpu.VMEM((NTOK,), jnp.int32),
        pltpu.VMEM((NTOK,), jnp.int32),
        pltpu.VMEM((H, W), jnp.float32),
        pltpu.VMEM((NTOK,), jnp.float32),
    ],
)
def _sc_gather(x_hbm, topk_hbm, tok_hbm, topk_v, posi_v, posj_v, plane_v,
               tok_v):
    wid = lax.axis_index("s") * NC + lax.axis_index("c")

    @pl.loop(0, NBN)
    def _bn_loop(bn):
        pltpu.sync_copy(topk_hbm.at[bn], topk_v)
        _compute_positions(topk_v, posi_v, posj_v)

        @pl.loop(0, DPW)
        def _plane_loop(local):
            b, ch = _plane_coords(bn, wid, local)
            d = wid * DPW + local
            pltpu.sync_copy(x_hbm.at[b, ch], plane_v)

            @pl.loop(0, NTOK // 16)
            def _tok_loop(c2):
                iv = posi_v[pl.ds(c2 * 16, 16)]
                jv = posj_v[pl.ds(c2 * 16, 16)]
                tok_v[pl.ds(c2 * 16, 16)] = plsc.load_gather(
                    plane_v, [iv, jv]
                )

            pltpu.sync_copy(tok_v, tok_hbm.at[bn, d])


@functools.partial(
    pl.kernel,
    out_type=jax.ShapeDtypeStruct((B, C, H, W), jnp.float32),
    mesh=_MESH,
    compiler_params=_SC_PARAMS,
    scratch_types=[
        pltpu.VMEM((KSEL,), jnp.int32),
        pltpu.VMEM((NTOK,), jnp.int32),
        pltpu.VMEM((NTOK,), jnp.int32),
        pltpu.VMEM((H, W), jnp.float32),
        pltpu.VMEM((NTOK,), jnp.float32),
    ],
)
def _sc_scatter(outtok_hbm, topk_hbm, out_hbm, topk_v, posi_v, posj_v,
                plane_v, tok_v):
    wid = lax.axis_index("s") * NC + lax.axis_index("c")

    @pl.loop(0, H)
    def _zero_loop(r):
        @pl.loop(0, W // 16)
        def _zero_row(ci):
            plane_v[r, pl.ds(ci * 16, 16)] = jnp.zeros((16,), jnp.float32)

    @pl.loop(0, NBN)
    def _bn_loop(bn):
        pltpu.sync_copy(topk_hbm.at[bn], topk_v)
        _compute_positions(topk_v, posi_v, posj_v)

        @pl.loop(0, DPW)
        def _plane_loop(local):
            b, ch = _plane_coords(bn, wid, local)
            d = wid * DPW + local
            pltpu.sync_copy(outtok_hbm.at[bn, d], tok_v)

            @pl.loop(0, NTOK // 16)
            def _scat_loop(c2):
                iv = posi_v[pl.ds(c2 * 16, 16)]
                jv = posj_v[pl.ds(c2 * 16, 16)]
                plsc.store_scatter(
                    plane_v, [iv, jv], tok_v[pl.ds(c2 * 16, 16)]
                )

            pltpu.sync_copy(plane_v, out_hbm.at[b, ch])

            @pl.loop(0, NTOK // 16)
            def _restore_loop(c2):
                iv = posi_v[pl.ds(c2 * 16, 16)]
                jv = posj_v[pl.ds(c2 * 16, 16)]
                plsc.store_scatter(
                    plane_v, [iv, jv], jnp.zeros((16,), jnp.float32)
                )


def _attn_body(tok_ref, w_ref, b_ref, out_ref):
    x = tok_ref[0]
    wq = w_ref[...]
    bias = b_ref[...]
    qkv = jnp.dot(wq, x, preferred_element_type=jnp.float32) + bias
    q = qkv[0:HD]
    k = qkv[HD:2 * HD]
    v = qkv[2 * HD:3 * HD]
    logits = lax.dot_general(
        q, k, (((0,), (0,)), ((), ())), preferred_element_type=jnp.float32
    ) * SCALE
    m = jnp.max(logits, axis=1, keepdims=True)
    p = jnp.exp(logits - m)
    s = jnp.sum(p, axis=1, keepdims=True)
    attn = p / s
    out_ref[0] = lax.dot_general(
        v, attn, (((1,), (1,)), ((), ())), preferred_element_type=jnp.float32
    )


_attn = pl.pallas_call(
    _attn_body,
    grid=(NBN,),
    in_specs=[
        pl.BlockSpec((1, HD, NTOK), lambda i: (i, 0, 0)),
        pl.BlockSpec((3 * HD, HD), lambda i: (0, 0)),
        pl.BlockSpec((3 * HD, 1), lambda i: (0, 0)),
    ],
    out_specs=pl.BlockSpec((1, HD, NTOK), lambda i: (i, 0, 0)),
    out_shape=jax.ShapeDtypeStruct((NBN, HD, NTOK), jnp.float32),
)


def kernel(x, top_k, Wqkv, bqkv):
    tk = top_k.reshape(NBN, KSEL)
    toks = _sc_gather(x, tk)
    out_t = _attn(toks, Wqkv, bqkv.reshape(3 * HD, 1))
    return _sc_scatter(out_t, tk)
